# Initial kernel scaffold; baseline (speedup 1.0000x reference)
import jax
import jax.numpy as jnp
from jax.experimental import pallas as pl


def kernel(x, edge_index, A_values):
    raise NotImplementedError("write your pallas kernel here")


# speedup vs baseline: 2.0761x; 2.0761x over previous
import functools

import jax
import jax.numpy as jnp
from jax import lax
from jax.experimental import pallas as pl
from jax.experimental.pallas import tpu as pltpu
from jax.experimental.pallas import tpu_sc as plsc

_N = 10000
_E = 320000
_D = 128
_HALF = _D // 2
_NS = 16
_EPT = _E // _NS
_B = 80
_NIT = _EPT // _B
_NPAD = 10240
_RPT = _NPAD // _NS


def _body(xc_hbm, src_hbm, dst_hbm, a_hbm, out_hbm,
          sidx, didx, av, rows, zbuf, shared, sem):
    c = lax.axis_index("c")
    s = lax.axis_index("s")

    zero = jnp.zeros((16,), jnp.float32)

    def zrow(i, carry):
        for cc in range(_HALF // 16):
            zbuf[i, pl.ds(cc * 16, 16)] = zero
        return carry

    lax.fori_loop(0, _RPT, zrow, 0)
    pltpu.sync_copy(zbuf, shared.at[pl.ds(s * _RPT, _RPT)])
    plsc.subcore_barrier()

    rowoff = c * _N

    def it(g, carry):
        base = s * _EPT + g * _B
        pltpu.sync_copy(src_hbm.at[pl.ds(base, _B)], sidx)
        pltpu.sync_copy(dst_hbm.at[pl.ds(base, _B)], didx)
        pltpu.sync_copy(a_hbm.at[pl.ds(base, _B)], av)
        for gg in range(_B // 16):
            sl = pl.ds(gg * 16, 16)
            sidx[sl] = sidx[sl] + rowoff
        pltpu.async_copy(xc_hbm.at[sidx], rows, sem).wait()

        def grp(gg, carry2):
            a16 = av[pl.ds(gg * 16, 16)]
            for j in range(16):
                r = gg * 16 + j
                avj = jnp.full((16,), a16[j], jnp.float32)
                for cc in range(_HALF // 16):
                    sl = pl.ds(cc * 16, 16)
                    rows[r, sl] = rows[r, sl] * avj
            return carry2

        lax.fori_loop(0, _B // 16, grp, 0)
        pltpu.sync_copy(rows, shared.at[didx], add=True)
        return carry

    lax.fori_loop(0, _NIT, it, 0)
    plsc.subcore_barrier()
    pltpu.sync_copy(shared.at[pl.ds(s * _RPT, _RPT)],
                    out_hbm.at[pl.ds(c * _NPAD + s * _RPT, _RPT)])


@jax.jit
def _propagate(x_cat, src, dst, a):
    mesh = plsc.VectorSubcoreMesh(core_axis_name="c", subcore_axis_name="s",
                                  num_cores=2, num_subcores=_NS)
    k = pl.kernel(
        _body,
        out_type=jax.ShapeDtypeStruct((2 * _NPAD, _HALF), jnp.float32),
        mesh=mesh,
        scratch_types=[
            pltpu.VMEM((_B,), jnp.int32),
            pltpu.VMEM((_B,), jnp.int32),
            pltpu.VMEM((_B,), jnp.float32),
            pltpu.VMEM((_B, _HALF), jnp.float32),
            pltpu.VMEM((_RPT, _HALF), jnp.float32),
            pltpu.VMEM_SHARED((_NPAD, _HALF), jnp.float32),
            pltpu.SemaphoreType.DMA,
        ],
        compiler_params=pltpu.CompilerParams(use_tc_tiling_on_sc=False),
    )
    return k(x_cat, src, dst, a)


def kernel(x, edge_index, A_values):
    x_cat = jnp.concatenate([x[:, :_HALF], x[:, _HALF:]], axis=0)
    out2 = _propagate(x_cat, edge_index[0], edge_index[1], A_values)
    return jnp.concatenate([out2[:_N], out2[_NPAD:_NPAD + _N]], axis=1)


# speedup vs baseline: 3.5133x; 1.6923x over previous
import jax
import jax.numpy as jnp
from jax import lax
from jax.experimental import pallas as pl
from jax.experimental.pallas import tpu as pltpu
from jax.experimental.pallas import tpu_sc as plsc

_N = 10000
_E = 320000
_D = 128
_HALF = _D // 2
_NS = 16
_EPT = _E // _NS
_B = 80
_NIT = _EPT // _B
_NPAD = 10240
_RPT = _NPAD // _NS


def _body(xc_hbm, src_hbm, dst_hbm, a_hbm, out_hbm,
          sidx, didx, av, rows, zbuf, shared, isem, gsem):
    c = lax.axis_index("c")
    s = lax.axis_index("s")

    zero = jnp.zeros((16,), jnp.float32)

    def zrow(i, carry):
        for cc in range(_HALF // 16):
            zbuf[i, pl.ds(cc * 16, 16)] = zero
        return carry

    lax.fori_loop(0, _RPT, zrow, 0)
    pltpu.sync_copy(zbuf, shared.at[pl.ds(s * _RPT, _RPT)])
    plsc.subcore_barrier()

    rowoff = c * _N
    ebase = s * _EPT

    def issue_idx(it, p):
        base = ebase + it * _B
        pltpu.async_copy(src_hbm.at[pl.ds(base, _B)], sidx.at[p], isem.at[p])
        pltpu.async_copy(dst_hbm.at[pl.ds(base, _B)], didx.at[p], isem.at[p])
        pltpu.async_copy(a_hbm.at[pl.ds(base, _B)], av.at[p], isem.at[p])

    def wait_idx(it, p):
        base = ebase + it * _B
        pltpu.make_async_copy(src_hbm.at[pl.ds(base, _B)], sidx.at[p],
                              isem.at[p]).wait()
        pltpu.make_async_copy(dst_hbm.at[pl.ds(base, _B)], didx.at[p],
                              isem.at[p]).wait()
        pltpu.make_async_copy(a_hbm.at[pl.ds(base, _B)], av.at[p],
                              isem.at[p]).wait()

    def adjust_and_gather(p):
        for gg in range(_B // 16):
            sl = pl.ds(gg * 16, 16)
            sidx[p, sl] = sidx[p, sl] + rowoff
        pltpu.async_copy(xc_hbm.at[sidx.at[p]], rows.at[p], gsem.at[p])

    def wait_gather(p):
        pltpu.make_async_copy(xc_hbm.at[sidx.at[p]], rows.at[p],
                              gsem.at[p]).wait()

    issue_idx(0, 0)
    wait_idx(0, 0)
    adjust_and_gather(0)
    issue_idx(1, 1)

    def step(it, p):
        @pl.when(it < _NIT - 1)
        def _():
            wait_idx(it + 1, 1 - p)

        wait_gather(p)

        @pl.when(it < _NIT - 1)
        def _():
            adjust_and_gather(1 - p)

        def grp(gg, carry2):
            a16 = av[p, pl.ds(gg * 16, 16)]
            for j in range(16):
                avj = jnp.full((16,), a16[j], jnp.float32)
                for cc in range(_HALF // 16):
                    sl = pl.ds(cc * 16, 16)
                    rows[p, gg * 16 + j, sl] = rows[p, gg * 16 + j, sl] * avj
            return carry2

        lax.fori_loop(0, _B // 16, grp, 0)
        pltpu.sync_copy(rows.at[p], shared.at[didx.at[p]], add=True)

        @pl.when(it < _NIT - 2)
        def _():
            issue_idx(it + 2, p)

    def outer(g, carry):
        step(2 * g, 0)
        step(2 * g + 1, 1)
        return carry

    lax.fori_loop(0, _NIT // 2, outer, 0)
    plsc.subcore_barrier()
    pltpu.sync_copy(shared.at[pl.ds(s * _RPT, _RPT)],
                    out_hbm.at[pl.ds(c * _NPAD + s * _RPT, _RPT)])


@jax.jit
def _propagate(x_cat, src, dst, a):
    mesh = plsc.VectorSubcoreMesh(core_axis_name="c", subcore_axis_name="s",
                                  num_cores=2, num_subcores=_NS)
    k = pl.kernel(
        _body,
        out_type=jax.ShapeDtypeStruct((2 * _NPAD, _HALF), jnp.float32),
        mesh=mesh,
        scratch_types=[
            pltpu.VMEM((2, _B), jnp.int32),
            pltpu.VMEM((2, _B), jnp.int32),
            pltpu.VMEM((2, _B), jnp.float32),
            pltpu.VMEM((2, _B, _HALF), jnp.float32),
            pltpu.VMEM((_RPT, _HALF), jnp.float32),
            pltpu.VMEM_SHARED((_NPAD, _HALF), jnp.float32),
            pltpu.SemaphoreType.DMA((2,)),
            pltpu.SemaphoreType.DMA((2,)),
        ],
        compiler_params=pltpu.CompilerParams(use_tc_tiling_on_sc=False),
    )
    return k(x_cat, src, dst, a)


def kernel(x, edge_index, A_values):
    x_cat = jnp.concatenate([x[:, :_HALF], x[:, _HALF:]], axis=0)
    out2 = _propagate(x_cat, edge_index[0], edge_index[1], A_values)
    return jnp.concatenate([out2[:_N], out2[_NPAD:_NPAD + _N]], axis=1)


# speedup vs baseline: 3.8913x; 1.1076x over previous
import jax
import jax.numpy as jnp
from jax import lax
from jax.experimental import pallas as pl
from jax.experimental.pallas import tpu as pltpu
from jax.experimental.pallas import tpu_sc as plsc

_N = 10000
_E = 320000
_D = 128
_HALF = _D // 2
_NS = 16
_B = 80
_NIT = 256
_EPT = _NIT * _B
_EPAD = _NS * _EPT
_NPAD = 10240
_RPT = _NPAD // _NS
_NROW = 4
_NIDX = 8


def _body(xc_hbm, src_hbm, dst_hbm, a_hbm, out_hbm,
          sidx, didx, av, rows, zbuf, shared, isem, gsem, ssem):
    c = lax.axis_index("c")
    s = lax.axis_index("s")

    zero = jnp.zeros((16,), jnp.float32)

    def zrow(i, carry):
        for cc in range(_HALF // 16):
            zbuf[i, pl.ds(cc * 16, 16)] = zero
        return carry

    lax.fori_loop(0, _RPT, zrow, 0)
    pltpu.sync_copy(zbuf, shared.at[pl.ds(s * _RPT, _RPT)])
    plsc.subcore_barrier()

    rowoff = c * _N
    ebase = s * _EPT

    def issue_idx(it, q):
        base = ebase + it * _B
        pltpu.async_copy(src_hbm.at[pl.ds(base, _B)], sidx.at[q], isem.at[q])
        pltpu.async_copy(dst_hbm.at[pl.ds(base, _B)], didx.at[q], isem.at[q])
        pltpu.async_copy(a_hbm.at[pl.ds(base, _B)], av.at[q], isem.at[q])

    def wait_idx(it, q):
        base = ebase + it * _B
        pltpu.make_async_copy(src_hbm.at[pl.ds(base, _B)], sidx.at[q],
                              isem.at[q]).wait()
        pltpu.make_async_copy(dst_hbm.at[pl.ds(base, _B)], didx.at[q],
                              isem.at[q]).wait()
        pltpu.make_async_copy(a_hbm.at[pl.ds(base, _B)], av.at[q],
                              isem.at[q]).wait()

    def adjust_and_gather(q, p):
        for gg in range(_B // 16):
            sl = pl.ds(gg * 16, 16)
            sidx[q, sl] = sidx[q, sl] + rowoff
        pltpu.async_copy(xc_hbm.at[sidx.at[q]], rows.at[p], gsem.at[p])

    def wait_gather(q, p):
        pltpu.make_async_copy(xc_hbm.at[sidx.at[q]], rows.at[p],
                              gsem.at[p]).wait()

    def issue_scatter(q, p):
        pltpu.async_copy(rows.at[p], shared.at[didx.at[q]], ssem.at[p],
                         add=True)

    def wait_scatter(q, p):
        pltpu.make_async_copy(rows.at[p], shared.at[didx.at[q]],
                              ssem.at[p]).wait()

    for k in range(4):
        issue_idx(k, k)
    for k in range(2):
        wait_idx(k, k)
        adjust_and_gather(k, k)

    def step(it, jj):
        q = jj % _NIDX
        p = jj % _NROW
        q2 = (jj + 2) % _NIDX
        p2 = (jj + 2) % _NROW

        wait_gather(q, p)

        @pl.when(it + 2 < _NIT)
        def _():
            wait_idx(it + 2, q2)

            @pl.when(it >= 2)
            def _():
                wait_scatter((jj - 2) % _NIDX, p2)

            adjust_and_gather(q2, p2)

        def grp(gg, carry2):
            a16 = av[q, pl.ds(gg * 16, 16)]
            for j in range(16):
                avj = jnp.full((16,), a16[j], jnp.float32)
                for cc in range(_HALF // 16):
                    sl = pl.ds(cc * 16, 16)
                    rows[p, gg * 16 + j, sl] = rows[p, gg * 16 + j, sl] * avj
            return carry2

        lax.fori_loop(0, _B // 16, grp, 0)
        issue_scatter(q, p)

        @pl.when(it + 4 < _NIT)
        def _():
            issue_idx(it + 4, (jj + 4) % _NIDX)

    def outer(g, carry):
        for jj in range(8):
            step(8 * g + jj, jj)
        return carry

    lax.fori_loop(0, _NIT // 8, outer, 0)

    for k in range(4):
        it = _NIT - 4 + k
        wait_scatter(it % _NIDX, it % _NROW)

    plsc.subcore_barrier()
    pltpu.sync_copy(shared.at[pl.ds(s * _RPT, _RPT)],
                    out_hbm.at[pl.ds(c * _NPAD + s * _RPT, _RPT)])


@jax.jit
def _propagate(x_cat, src, dst, a):
    mesh = plsc.VectorSubcoreMesh(core_axis_name="c", subcore_axis_name="s",
                                  num_cores=2, num_subcores=_NS)
    k = pl.kernel(
        _body,
        out_type=jax.ShapeDtypeStruct((2 * _NPAD, _HALF), jnp.float32),
        mesh=mesh,
        scratch_types=[
            pltpu.VMEM((_NIDX, _B), jnp.int32),
            pltpu.VMEM((_NIDX, _B), jnp.int32),
            pltpu.VMEM((_NIDX, _B), jnp.float32),
            pltpu.VMEM((_NROW, _B, _HALF), jnp.float32),
            pltpu.VMEM((_RPT, _HALF), jnp.float32),
            pltpu.VMEM_SHARED((_NPAD, _HALF), jnp.float32),
            pltpu.SemaphoreType.DMA((_NIDX,)),
            pltpu.SemaphoreType.DMA((_NROW,)),
            pltpu.SemaphoreType.DMA((_NROW,)),
        ],
        compiler_params=pltpu.CompilerParams(use_tc_tiling_on_sc=False),
    )
    return k(x_cat, src, dst, a)


def kernel(x, edge_index, A_values):
    x_cat = jnp.concatenate([x[:, :_HALF], x[:, _HALF:]], axis=0)
    pad = _EPAD - _E
    src = jnp.pad(edge_index[0], (0, pad))
    dst = jnp.pad(edge_index[1], (0, pad))
    a = jnp.pad(A_values, (0, pad))
    out2 = _propagate(x_cat, src, dst, a)
    return jnp.concatenate([out2[:_N], out2[_NPAD:_NPAD + _N]], axis=1)


# speedup vs baseline: 4.6938x; 1.2063x over previous
import jax
import jax.numpy as jnp
from jax import lax
from jax.experimental import pallas as pl
from jax.experimental.pallas import tpu as pltpu
from jax.experimental.pallas import tpu_sc as plsc

_N = 10000
_E = 320000
_D = 128
_HALF = _D // 2
_NS = 16
_B = 128
_NIT = 160
_EPT = _NIT * _B
_EPAD = _NS * _EPT
_NPAD = 10240
_RPT = _NPAD // _NS
_NROW = 4
_NIDX = 8


def _body(xc_hbm, epk_hbm, out_hbm, ebuf, rows, zbuf, shared,
          isem, gsem, ssem):
    c = lax.axis_index("c")
    s = lax.axis_index("s")

    zero = jnp.zeros((16,), jnp.float32)

    def zrow(i, carry):
        for cc in range(_HALF // 16):
            zbuf[i, pl.ds(cc * 16, 16)] = zero
        return carry

    lax.fori_loop(0, _RPT, zrow, 0)
    pltpu.sync_copy(zbuf, shared.at[pl.ds(s * _RPT, _RPT)])
    plsc.subcore_barrier()

    rowoff = c * _N
    bbase = s * _NIT

    def issue_idx(it, q):
        pltpu.async_copy(epk_hbm.at[bbase + it], ebuf.at[q], isem.at[q])

    def wait_idx(it, q):
        pltpu.make_async_copy(epk_hbm.at[bbase + it], ebuf.at[q],
                              isem.at[q]).wait()

    def adjust_and_gather(q, p):
        for gg in range(_B // 16):
            sl = pl.ds(gg * 16, 16)
            ebuf[q, 0, sl] = ebuf[q, 0, sl] + rowoff
        pltpu.async_copy(xc_hbm.at[ebuf.at[q, 0]], rows.at[p], gsem.at[p])

    def wait_gather(q, p):
        pltpu.make_async_copy(xc_hbm.at[ebuf.at[q, 0]], rows.at[p],
                              gsem.at[p]).wait()

    def issue_scatter(q, p):
        pltpu.async_copy(rows.at[p], shared.at[ebuf.at[q, 1]], ssem.at[p],
                         add=True)

    def wait_scatter(q, p):
        pltpu.make_async_copy(rows.at[p], shared.at[ebuf.at[q, 1]],
                              ssem.at[p]).wait()

    for k in range(4):
        issue_idx(k, k)
    for k in range(2):
        wait_idx(k, k)
        adjust_and_gather(k, k)

    def step(it, jj):
        q = jj % _NIDX
        p = jj % _NROW
        q2 = (jj + 2) % _NIDX
        p2 = (jj + 2) % _NROW

        wait_gather(q, p)

        @pl.when(it + 2 < _NIT)
        def _():
            wait_idx(it + 2, q2)

            @pl.when(it >= 2)
            def _():
                wait_scatter((jj - 2) % _NIDX, p2)

            adjust_and_gather(q2, p2)

        def grp(gg, carry2):
            a16 = plsc.bitcast(ebuf[q, 2, pl.ds(gg * 16, 16)], jnp.float32)
            for j in range(16):
                avj = jnp.full((16,), a16[j], jnp.float32)
                for cc in range(_HALF // 16):
                    sl = pl.ds(cc * 16, 16)
                    rows[p, gg * 16 + j, sl] = rows[p, gg * 16 + j, sl] * avj
            return carry2

        lax.fori_loop(0, _B // 16, grp, 0)
        issue_scatter(q, p)

        @pl.when(it + 4 < _NIT)
        def _():
            issue_idx(it + 4, (jj + 4) % _NIDX)

    def outer(g, carry):
        for jj in range(8):
            step(8 * g + jj, jj)
        return carry

    lax.fori_loop(0, _NIT // 8, outer, 0)

    for k in range(4):
        it = _NIT - 4 + k
        wait_scatter(it % _NIDX, it % _NROW)

    plsc.subcore_barrier()
    pltpu.sync_copy(shared.at[pl.ds(s * _RPT, _RPT)],
                    out_hbm.at[pl.ds(c * _NPAD + s * _RPT, _RPT)])


@jax.jit
def _propagate(x_cat, epk):
    mesh = plsc.VectorSubcoreMesh(core_axis_name="c", subcore_axis_name="s",
                                  num_cores=2, num_subcores=_NS)
    k = pl.kernel(
        _body,
        out_type=jax.ShapeDtypeStruct((2 * _NPAD, _HALF), jnp.float32),
        mesh=mesh,
        scratch_types=[
            pltpu.VMEM((_NIDX, 3, _B), jnp.int32),
            pltpu.VMEM((_NROW, _B, _HALF), jnp.float32),
            pltpu.VMEM((_RPT, _HALF), jnp.float32),
            pltpu.VMEM_SHARED((_NPAD, _HALF), jnp.float32),
            pltpu.SemaphoreType.DMA((_NIDX,)),
            pltpu.SemaphoreType.DMA((_NROW,)),
            pltpu.SemaphoreType.DMA((_NROW,)),
        ],
        compiler_params=pltpu.CompilerParams(use_tc_tiling_on_sc=False,
                                             needs_layout_passes=False),
    )
    return k(x_cat, epk)


def kernel(x, edge_index, A_values):
    x_cat = jnp.concatenate([x[:, :_HALF], x[:, _HALF:]], axis=0)
    pad = _EPAD - _E
    src = jnp.pad(edge_index[0], (0, pad)).reshape(_NS * _NIT, _B)
    dst = jnp.pad(edge_index[1], (0, pad)).reshape(_NS * _NIT, _B)
    a = lax.bitcast_convert_type(jnp.pad(A_values, (0, pad)),
                                 jnp.int32).reshape(_NS * _NIT, _B)
    epk = jnp.stack([src, dst, a], axis=1)
    out2 = _propagate(x_cat, epk)
    return jnp.concatenate([out2[:_N], out2[_NPAD:_NPAD + _N]], axis=1)


# speedup vs baseline: 6.3039x; 1.3430x over previous
import jax
import jax.numpy as jnp
from jax import lax
from jax.experimental import pallas as pl
from jax.experimental.pallas import tpu as pltpu
from jax.experimental.pallas import tpu_sc as plsc

_N = 10000
_E = 320000
_D = 128
_HALF = _D // 2
_NS = 16
_B = 128
_NIT = 160
_EPT = _NIT * _B
_EPAD = _NS * _EPT
_NPAD = 10240
_RPT = _NPAD // _NS
_ZR = 128
_NROW = 4
_NIDX = 8


def _body(xc_hbm, epk_hbm, out_hbm, ebuf, rows, zbuf, shared, xs,
          isem, gsem, ssem):
    c = lax.axis_index("c")
    s = lax.axis_index("s")

    zero = jnp.zeros((16,), jnp.float32)

    def zrow(i, carry):
        for cc in range(_HALF // 16):
            zbuf[i, pl.ds(cc * 16, 16)] = zero
        return carry

    lax.fori_loop(0, _ZR, zrow, 0)
    for k in range(_RPT // _ZR):
        pltpu.sync_copy(zbuf, shared.at[pl.ds(s * _RPT + k * _ZR, _ZR)])
    pltpu.sync_copy(xc_hbm.at[pl.ds(c * _NPAD + s * _RPT, _RPT)],
                    xs.at[pl.ds(s * _RPT, _RPT)])
    plsc.subcore_barrier()

    bbase = s * _NIT

    def issue_idx(it, q):
        pltpu.async_copy(epk_hbm.at[bbase + it], ebuf.at[q], isem.at[q])

    def wait_idx(it, q):
        pltpu.make_async_copy(epk_hbm.at[bbase + it], ebuf.at[q],
                              isem.at[q]).wait()

    def adjust_and_gather(q, p):
        pltpu.async_copy(xs.at[ebuf.at[q, 0]], rows.at[p], gsem.at[p])

    def wait_gather(q, p):
        pltpu.make_async_copy(xs.at[ebuf.at[q, 0]], rows.at[p],
                              gsem.at[p]).wait()

    def issue_scatter(q, p):
        pltpu.async_copy(rows.at[p], shared.at[ebuf.at[q, 1]], ssem.at[p],
                         add=True)

    def wait_scatter(q, p):
        pltpu.make_async_copy(rows.at[p], shared.at[ebuf.at[q, 1]],
                              ssem.at[p]).wait()

    for k in range(4):
        issue_idx(k, k)
    for k in range(2):
        wait_idx(k, k)
        adjust_and_gather(k, k)

    def step(it, jj):
        q = jj % _NIDX
        p = jj % _NROW
        q2 = (jj + 2) % _NIDX
        p2 = (jj + 2) % _NROW

        wait_gather(q, p)

        @pl.when(it + 2 < _NIT)
        def _():
            wait_idx(it + 2, q2)

            @pl.when(it >= 2)
            def _():
                wait_scatter((jj - 2) % _NIDX, p2)

            adjust_and_gather(q2, p2)

        def grp(gg, carry2):
            a16 = plsc.bitcast(ebuf[q, 2, pl.ds(gg * 16, 16)], jnp.float32)
            for j in range(16):
                avj = jnp.full((16,), a16[j], jnp.float32)
                for cc in range(_HALF // 16):
                    sl = pl.ds(cc * 16, 16)
                    rows[p, gg * 16 + j, sl] = rows[p, gg * 16 + j, sl] * avj
            return carry2

        lax.fori_loop(0, _B // 16, grp, 0)
        issue_scatter(q, p)

        @pl.when(it + 4 < _NIT)
        def _():
            issue_idx(it + 4, (jj + 4) % _NIDX)

    def outer(g, carry):
        for jj in range(8):
            step(8 * g + jj, jj)
        return carry

    lax.fori_loop(0, _NIT // 8, outer, 0)

    for k in range(4):
        it = _NIT - 4 + k
        wait_scatter(it % _NIDX, it % _NROW)

    plsc.subcore_barrier()
    pltpu.sync_copy(shared.at[pl.ds(s * _RPT, _RPT)],
                    out_hbm.at[pl.ds(c * _NPAD + s * _RPT, _RPT)])


@jax.jit
def _propagate(x_cat, epk):
    mesh = plsc.VectorSubcoreMesh(core_axis_name="c", subcore_axis_name="s",
                                  num_cores=2, num_subcores=_NS)
    k = pl.kernel(
        _body,
        out_type=jax.ShapeDtypeStruct((2 * _NPAD, _HALF), jnp.float32),
        mesh=mesh,
        scratch_types=[
            pltpu.VMEM((_NIDX, 3, _B), jnp.int32),
            pltpu.VMEM((_NROW, _B, _HALF), jnp.float32),
            pltpu.VMEM((_ZR, _HALF), jnp.float32),
            pltpu.VMEM_SHARED((_NPAD, _HALF), jnp.float32),
            pltpu.VMEM_SHARED((_NPAD, _HALF), jnp.float32),
            pltpu.SemaphoreType.DMA((_NIDX,)),
            pltpu.SemaphoreType.DMA((_NROW,)),
            pltpu.SemaphoreType.DMA((_NROW,)),
        ],
        compiler_params=pltpu.CompilerParams(use_tc_tiling_on_sc=False,
                                             needs_layout_passes=False),
    )
    return k(x_cat, epk)


def kernel(x, edge_index, A_values):
    xp = jnp.pad(x, ((0, _NPAD - _N), (0, 0)))
    x_cat = jnp.concatenate([xp[:, :_HALF], xp[:, _HALF:]], axis=0)
    pad = _EPAD - _E
    src = jnp.pad(edge_index[0], (0, pad)).reshape(_NS * _NIT, _B)
    dst = jnp.pad(edge_index[1], (0, pad)).reshape(_NS * _NIT, _B)
    a = lax.bitcast_convert_type(jnp.pad(A_values, (0, pad)),
                                 jnp.int32).reshape(_NS * _NIT, _B)
    epk = jnp.stack([src, dst, a], axis=1)
    out2 = _propagate(x_cat, epk)
    return jnp.concatenate([out2[:_N], out2[_NPAD:_NPAD + _N]], axis=1)
